# baseline (speedup 1.0000x reference)
import jax
import jax.numpy as jnp
from jax.experimental import pallas as pl
from jax.experimental.pallas import tpu as pltpu

_VOCAB = 1_000_000
_EMB = 64
_ROWS = 500_000
_COLS = 128
_BLOCK_ROWS = 4_000


def _copy_body(in_ref, out_ref):
    out_ref[...] = in_ref[...]


def kernel(lang, W_emb):
    del lang
    W2 = W_emb.reshape(_ROWS, _COLS)
    out = pl.pallas_call(
        _copy_body,
        grid=(_ROWS // _BLOCK_ROWS,),
        in_specs=[pl.BlockSpec((_BLOCK_ROWS, _COLS), lambda i: (i, 0))],
        out_specs=pl.BlockSpec((_BLOCK_ROWS, _COLS), lambda i: (i, 0)),
        out_shape=jax.ShapeDtypeStruct((_ROWS, _COLS), jnp.float32),
        compiler_params=pltpu.CompilerParams(
            dimension_semantics=("parallel",),
        ),
    )(W2)
    return out.reshape(_VOCAB, _EMB)


# speedup vs baseline: 1.3101x; 1.3101x over previous
import functools

import jax
import jax.numpy as jnp
from jax import lax
from jax.experimental import pallas as pl
from jax.experimental.pallas import tpu as pltpu
from jax.experimental.pallas import tpu_sc as plsc

_VOCAB = 1_000_000
_EMB = 64
_NWORKERS = 32
_CHUNK = 504
_NCH = 62
_STRIPE = _CHUNK * _NCH
_TAIL = _VOCAB - _NWORKERS * _STRIPE
_TAIL_BASE = _NWORKERS * _STRIPE


def _sc_body(in_hbm, out_hbm, buf, in_sems, out_sems):
    wid = lax.axis_index("s") * 2 + lax.axis_index("c")
    base = wid * _STRIPE

    def rows(i):
        return pl.ds(pl.multiple_of(base + i * _CHUNK, 8), _CHUNK)

    def in_copy(i, slot):
        return pltpu.make_async_copy(
            in_hbm.at[rows(i), :], buf.at[slot], in_sems.at[slot]
        )

    def out_copy(i, slot):
        return pltpu.make_async_copy(
            buf.at[slot], out_hbm.at[rows(i), :], out_sems.at[slot]
        )

    in_copy(0, 0).start()
    in_copy(0, 0).wait()
    out_copy(0, 0).start()
    in_copy(1, 1).start()
    in_copy(1, 1).wait()
    out_copy(1, 1).start()
    out_copy(0, 0).wait()
    in_copy(2, 0).start()

    def pair(g, _):
        for b in (0, 1):
            i = 2 * g + b
            in_copy(i, b).wait()
            out_copy(i, b).start()
            out_copy(i - 1, 1 - b).wait()
            in_copy(i + 1, 1 - b).start()
        return ()

    lax.fori_loop(1, _NCH // 2 - 1, pair, (), unroll=1)

    i = _NCH - 2
    in_copy(i, 0).wait()
    out_copy(i, 0).start()
    out_copy(i - 1, 1).wait()
    in_copy(i + 1, 1).start()
    in_copy(i + 1, 1).wait()
    out_copy(i + 1, 1).start()
    out_copy(i, 0).wait()
    out_copy(i + 1, 1).wait()

    @pl.when(wid == _NWORKERS - 1)
    def _():
        t_in = pltpu.make_async_copy(
            in_hbm.at[pl.ds(_TAIL_BASE, _TAIL), :],
            buf.at[0, pl.ds(0, _TAIL), :],
            in_sems.at[0],
        )
        t_out = pltpu.make_async_copy(
            buf.at[0, pl.ds(0, _TAIL), :],
            out_hbm.at[pl.ds(_TAIL_BASE, _TAIL), :],
            out_sems.at[0],
        )
        t_in.start()
        t_in.wait()
        t_out.start()
        t_out.wait()


def _sc_copy(W_emb):
    mesh = plsc.VectorSubcoreMesh(core_axis_name="c", subcore_axis_name="s")
    k = functools.partial(
        pl.kernel,
        mesh=mesh,
        out_type=jax.ShapeDtypeStruct((_VOCAB, _EMB), jnp.float32),
        scratch_types=[
            pltpu.VMEM((2, _CHUNK, _EMB), jnp.float32),
            pltpu.SemaphoreType.DMA((2,)),
            pltpu.SemaphoreType.DMA((2,)),
        ],
    )(_sc_body)
    return k(W_emb)


def kernel(lang, W_emb):
    del lang
    return _sc_copy(W_emb)


# speedup vs baseline: 7.0272x; 5.3640x over previous
import functools

import jax
import jax.numpy as jnp
from jax import lax
from jax.experimental import pallas as pl
from jax.experimental.pallas import tpu as pltpu
from jax.experimental.pallas import tpu_sc as plsc

_VOCAB = 1_000_000
_EMB = 64
_NROWG = 8
_NCOLG = 4
_COLG = 249_984
_CHUNK = 3_968
_NCH = _COLG // _CHUNK
_TAIL_BASE = _NCOLG * _COLG
_TAIL = _VOCAB - _TAIL_BASE


def _sc_body(in_hbm, out_hbm, buf, tail_buf, in_sems, out_sems):
    wid = lax.axis_index("s") * 2 + lax.axis_index("c")
    rowg = wid // _NCOLG
    colg = wid % _NCOLG
    row0 = pl.multiple_of(rowg * _NROWG, 8)
    col_base = colg * _COLG

    def cols(i):
        return pl.ds(pl.multiple_of(col_base + i * _CHUNK, 128), _CHUNK)

    def in_copy(i, slot):
        return pltpu.make_async_copy(
            in_hbm.at[pl.ds(row0, _NROWG), cols(i)],
            buf.at[slot],
            in_sems.at[slot],
        )

    def out_copy(i, slot):
        return pltpu.make_async_copy(
            buf.at[slot],
            out_hbm.at[pl.ds(row0, _NROWG), cols(i)],
            out_sems.at[slot],
        )

    in_copy(0, 0).start()
    in_copy(0, 0).wait()
    out_copy(0, 0).start()
    in_copy(1, 1).start()
    in_copy(1, 1).wait()
    out_copy(1, 1).start()
    out_copy(0, 0).wait()
    in_copy(2, 0).start()

    def pair(g, _):
        for b in (0, 1):
            i = 2 * g + b
            in_copy(i, b).wait()
            out_copy(i, b).start()
            out_copy(i - 1, 1 - b).wait()
            in_copy(i + 1, 1 - b).start()
        return ()

    lax.fori_loop(1, (_NCH - 3) // 2, pair, (), unroll=1)

    i = _NCH - 3
    in_copy(i, 0).wait()
    out_copy(i, 0).start()
    out_copy(i - 1, 1).wait()
    in_copy(i + 1, 1).start()
    in_copy(i + 1, 1).wait()
    out_copy(i + 1, 1).start()
    out_copy(i, 0).wait()
    in_copy(i + 2, 0).start()
    in_copy(i + 2, 0).wait()
    out_copy(i + 2, 0).start()
    out_copy(i + 1, 1).wait()
    out_copy(i + 2, 0).wait()

    @pl.when(wid == _NROWG * _NCOLG - 1)
    def _():
        t_in = pltpu.make_async_copy(
            in_hbm.at[:, pl.ds(_TAIL_BASE, _TAIL)], tail_buf, in_sems.at[0]
        )
        t_out = pltpu.make_async_copy(
            tail_buf, out_hbm.at[:, pl.ds(_TAIL_BASE, _TAIL)], out_sems.at[0]
        )
        t_in.start()
        t_in.wait()
        t_out.start()
        t_out.wait()


def _sc_copy(W_t):
    mesh = plsc.VectorSubcoreMesh(core_axis_name="c", subcore_axis_name="s")
    k = functools.partial(
        pl.kernel,
        mesh=mesh,
        out_type=jax.ShapeDtypeStruct((_EMB, _VOCAB), jnp.float32),
        scratch_types=[
            pltpu.VMEM((2, _NROWG, _CHUNK), jnp.float32),
            pltpu.VMEM((_EMB, _TAIL), jnp.float32),
            pltpu.SemaphoreType.DMA((2,)),
            pltpu.SemaphoreType.DMA((2,)),
        ],
    )(_sc_body)
    return k(W_t)


def kernel(lang, W_emb):
    del lang
    return _sc_copy(W_emb.T).T
